# Initial kernel scaffold; baseline (speedup 1.0000x reference)
import jax
import jax.numpy as jnp
from jax.experimental import pallas as pl


def kernel(samples, enc_weight, cent_weight):
    raise NotImplementedError("write your pallas kernel here")


# speedup vs baseline: 1.3982x; 1.3982x over previous
import jax
import jax.numpy as jnp
from jax.experimental import pallas as pl


def _fused_kernel(samples_ref, enc_ref, cent_ref, out_ref):
    t = jax.lax.dot_general(
        cent_ref[...], enc_ref[...],
        (((1,), (0,)), ((), ())),
        preferred_element_type=jnp.float32,
    )
    out_ref[...] = jax.lax.dot_general(
        samples_ref[...], t,
        (((1,), (1,)), ((), ())),
        preferred_element_type=jnp.float32,
    )


def kernel(samples, enc_weight, cent_weight):
    batch, n_features = samples.shape
    n_classes = cent_weight.shape[0]
    return pl.pallas_call(
        _fused_kernel,
        out_shape=jax.ShapeDtypeStruct((batch, n_classes), jnp.float32),
    )(samples, enc_weight, cent_weight)
